# baseline (speedup 1.0000x reference)
import jax
import jax.numpy as jnp
from jax import lax
from jax.experimental import pallas as pl
from jax.experimental.pallas import tpu as pltpu
from jax.experimental.pallas import tpu_sc as plsc

_PAD = 0
_MASK = 1
_B, _S, _D = 4096, 200, 64
_N = _B * _S
_NC, _NS, _L = 2, 16, 16
_NW = _NC * _NS
_ROWS_PER_W = _N // _NW
_SUB = 128
_CHUNK = 512
_NSUB = _CHUNK // _SUB
_STEPS = _ROWS_PER_W // _CHUNK
_GROUPS = _CHUNK // _L


def _group_ids(ids2d_v, g):
    row = jnp.zeros((_L,), jnp.int32) + g // (_SUB // _L)
    col = (g % (_SUB // _L)) * _L + lax.iota(jnp.int32, _L)
    return plsc.load_gather(ids2d_v, [row, col])


def _body(ids2d_hbm, table_hbm, pm_hbm, out_hbm,
          ids2d_v, rows_v, pm_v, gsem):
    wid = lax.axis_index("s") * _NC + lax.axis_index("c")
    chunk_base = wid * (_ROWS_PER_W // _SUB)
    row_base = wid * _ROWS_PER_W

    pltpu.sync_copy(pm_hbm, pm_v)

    def step(i, carry):
        pltpu.sync_copy(ids2d_hbm.at[pl.ds(chunk_base + i * _NSUB, _NSUB), :],
                        ids2d_v)
        copies = []
        for j in range(_NSUB):
            copies.append(
                pltpu.async_copy(table_hbm.at[ids2d_v.at[j]],
                                 rows_v.at[pl.ds(j * _SUB, _SUB), :], gsem))
        for c in copies:
            c.wait()

        def detect(g, acc):
            v_ids = _group_ids(ids2d_v, g)
            return acc | jnp.where(v_ids < 2, 1, 0).astype(jnp.int32)

        spec = lax.fori_loop(0, _GROUPS, detect,
                             jnp.zeros((_L,), jnp.int32))
        cnt = plsc.all_reduce_population_count(spec > 0)

        @pl.when(cnt[0] > 0)
        def _fixup():
            def fix_group(g, carry2):
                v_ids = _group_ids(ids2d_v, g)
                special = v_ids < 2
                sel_row = jnp.where(v_ids == _MASK, 1, 0).astype(jnp.int32)
                rows16 = g * _L + lax.iota(jnp.int32, _L)
                for d in range(_D):
                    dcol = jnp.full((_L,), d, jnp.int32)
                    val = plsc.load_gather(pm_v, [sel_row, dcol])
                    plsc.store_scatter(rows_v, [rows16, dcol], val,
                                       mask=special)
                return carry2

            lax.fori_loop(0, _GROUPS, fix_group, 0)

        pltpu.sync_copy(rows_v,
                        out_hbm.at[pl.ds(row_base + i * _CHUNK, _CHUNK), :])
        return carry

    lax.fori_loop(0, _STEPS, step, 0)


def kernel(road_ids, road_table, pad_emb, mask_emb):
    ids2d = road_ids.reshape(_N // _SUB, _SUB)
    pm = jnp.stack([pad_emb, mask_emb])

    mesh = plsc.VectorSubcoreMesh(core_axis_name="c", subcore_axis_name="s")
    run = pl.kernel(
        _body,
        mesh=mesh,
        compiler_params=pltpu.CompilerParams(needs_layout_passes=False,
                                             use_tc_tiling_on_sc=False),
        out_type=jax.ShapeDtypeStruct((_N, _D), jnp.float32),
        scratch_types=[
            pltpu.VMEM((_NSUB, _SUB), jnp.int32),
            pltpu.VMEM((_CHUNK, _D), jnp.float32),
            pltpu.VMEM((2, _D), jnp.float32),
            pltpu.SemaphoreType.DMA,
        ],
    )
    out = run(ids2d, road_table, pm)
    return out.reshape(_B, _S, _D)


# speedup vs baseline: 1.0504x; 1.0504x over previous
import jax
import jax.numpy as jnp
from jax import lax
from jax.experimental import pallas as pl
from jax.experimental.pallas import tpu as pltpu
from jax.experimental.pallas import tpu_sc as plsc

_PAD = 0
_MASK = 1
_B, _S, _D = 4096, 200, 64
_N = _B * _S
_NC, _NS, _L = 2, 16, 16
_NW = _NC * _NS
_ROWS_PER_W = _N // _NW
_SUB = 128
_CHUNK = 512
_NSUB = _CHUNK // _SUB
_STEPS = _ROWS_PER_W // _CHUNK
_GROUPS = _CHUNK // _L


def _group_ids(ids2d_v, g):
    row = jnp.zeros((_L,), jnp.int32) + g // (_SUB // _L)
    col = (g % (_SUB // _L)) * _L + lax.iota(jnp.int32, _L)
    return plsc.load_gather(ids2d_v, [row, col])


def _body(ids2d_hbm, table_hbm, pm_hbm, out_hbm,
          ids_v0, ids_v1, rows_v0, rows_v1, pm_v,
          ids_sem, gat_sem, out_sem0, out_sem1):
    ids_bufs = (ids_v0, ids_v1)
    rows_bufs = (rows_v0, rows_v1)
    out_sems = (out_sem0, out_sem1)

    wid = lax.axis_index("s") * _NC + lax.axis_index("c")
    chunk_base = wid * (_ROWS_PER_W // _SUB)
    row_base = wid * _ROWS_PER_W

    def ids_src(i):
        return ids2d_hbm.at[pl.ds(chunk_base + i * _NSUB, _NSUB), :]

    def out_dst(i):
        return out_hbm.at[pl.ds(row_base + i * _CHUNK, _CHUNK), :]

    pltpu.sync_copy(pm_hbm, pm_v)
    pltpu.async_copy(ids_src(0), ids_bufs[0], ids_sem)

    def pair(i2, carry):
        for b in range(2):
            i = 2 * i2 + b
            pltpu.make_async_copy(ids_src(i), ids_bufs[b], ids_sem).wait()

            @pl.when(i < _STEPS - 1)
            def _prefetch():
                pltpu.async_copy(ids_src(i + 1), ids_bufs[1 - b], ids_sem)

            @pl.when(i2 > 0)
            def _drain():
                pltpu.make_async_copy(rows_bufs[b], out_dst(i),
                                      out_sems[b]).wait()

            copies = []
            for j in range(_NSUB):
                copies.append(pltpu.async_copy(
                    table_hbm.at[ids_bufs[b].at[j]],
                    rows_bufs[b].at[pl.ds(j * _SUB, _SUB), :], gat_sem))

            def detect(g, acc):
                v_ids = _group_ids(ids_bufs[b], g)
                return acc | jnp.where(v_ids < 2, 1, 0).astype(jnp.int32)

            spec = lax.fori_loop(0, _GROUPS, detect,
                                 jnp.zeros((_L,), jnp.int32))
            cnt = plsc.all_reduce_population_count(spec > 0)

            for c in copies:
                c.wait()

            @pl.when(cnt[0] > 0)
            def _fixup():
                def fix_group(g, carry2):
                    v_ids = _group_ids(ids_bufs[b], g)
                    special = v_ids < 2
                    sel = jnp.where(v_ids == _MASK, 1, 0).astype(jnp.int32)
                    rows16 = g * _L + lax.iota(jnp.int32, _L)
                    for d in range(_D):
                        dcol = jnp.full((_L,), d, jnp.int32)
                        val = plsc.load_gather(pm_v, [sel, dcol])
                        plsc.store_scatter(rows_bufs[b], [rows16, dcol], val,
                                           mask=special)
                    return carry2

                lax.fori_loop(0, _GROUPS, fix_group, 0)

            pltpu.async_copy(rows_bufs[b], out_dst(i), out_sems[b])
        return carry

    lax.fori_loop(0, _STEPS // 2, pair, 0)

    for b in range(2):
        pltpu.make_async_copy(rows_bufs[b], out_dst(_STEPS - 2 + b),
                              out_sems[b]).wait()


def kernel(road_ids, road_table, pad_emb, mask_emb):
    ids2d = road_ids.reshape(_N // _SUB, _SUB)
    pm = jnp.stack([pad_emb, mask_emb])

    mesh = plsc.VectorSubcoreMesh(core_axis_name="c", subcore_axis_name="s")
    run = pl.kernel(
        _body,
        mesh=mesh,
        compiler_params=pltpu.CompilerParams(needs_layout_passes=False,
                                             use_tc_tiling_on_sc=False),
        out_type=jax.ShapeDtypeStruct((_N, _D), jnp.float32),
        scratch_types=[
            pltpu.VMEM((_NSUB, _SUB), jnp.int32),
            pltpu.VMEM((_NSUB, _SUB), jnp.int32),
            pltpu.VMEM((_CHUNK, _D), jnp.float32),
            pltpu.VMEM((_CHUNK, _D), jnp.float32),
            pltpu.VMEM((2, _D), jnp.float32),
            pltpu.SemaphoreType.DMA,
            pltpu.SemaphoreType.DMA,
            pltpu.SemaphoreType.DMA,
            pltpu.SemaphoreType.DMA,
        ],
    )
    out = run(ids2d, road_table, pm)
    return out.reshape(_B, _S, _D)
